# baseline (speedup 1.0000x reference)
import functools

import jax
import jax.numpy as jnp
from jax import lax
from jax.experimental import pallas as pl
from jax.experimental.pallas import tpu as pltpu
from jax.experimental.pallas import tpu_sc as plsc

B = 16384
R_OUT = 6
D = 128
LANES = 16

_info = plsc.get_sparse_core_info()
NC, NS = _info.num_cores, _info.num_subcores
NW = NC * NS
PER_W = B // NW
NCHUNK = 16
NROWS = NCHUNK * R_OUT
NSLOT = 4
NSTEPS = PER_W // NCHUNK
NGRP = NROWS // LANES


def _body(xf_hbm, out_hbm, bufs, ixs0, ixs1, bidx0, sgs, sss):
    wid = lax.axis_index("s") * NC + lax.axis_index("c")
    base = wid * PER_W

    lane = jax.lax.iota(jnp.int32, LANES)
    for g in range(NGRP):
        r = g * LANES + lane
        bidx0[pl.ds(g * LANES, LANES)] = (
            lax.rem(r, R_OUT) * B + lax.div(r, R_OUT))

    def set_idx(p, step):
        off = base + step * NCHUNK
        for g in range(NGRP):
            sl = pl.ds(g * LANES, LANES)
            v = bidx0[sl] + off
            ixs0[p][sl] = v
            ixs1[p][sl] = v + B

    def start_g0(p):
        return pltpu.async_copy(xf_hbm.at[ixs0[p]], bufs[p], sgs[p])

    def wait_g(p):
        pltpu.make_async_copy(xf_hbm.at[ixs0[p]], bufs[p], sgs[p]).wait()

    def start_ga(p):
        return pltpu.async_copy(xf_hbm.at[ixs1[p]], bufs[p], sgs[p], add=True)

    def start_sc(step, p):
        off6 = (base + step * NCHUNK) * R_OUT
        return pltpu.async_copy(
            bufs[p], out_hbm.at[pl.ds(off6, NROWS)], sss[p])

    def wait_sc(p):
        pltpu.make_async_copy(
            bufs[p], out_hbm.at[pl.ds(0, NROWS)], sss[p]).wait()

    for p in range(NSLOT):
        set_idx(p, p)
        start_g0(p)

    def quad_body(q, carry):
        s0 = NSLOT * q
        for p in range(NSLOT):
            wait_g(p)
            start_ga(p)
        for p in range(NSLOT):
            wait_g(p)
            start_sc(s0 + p, p)
        for p in range(NSLOT):
            @pl.when(s0 + NSLOT + p < NSTEPS)
            def _(p=p):
                wait_sc(p)
                set_idx(p, s0 + NSLOT + p)
                start_g0(p)
        return carry

    lax.fori_loop(0, NSTEPS // NSLOT, quad_body, 0)
    for p in range(NSLOT):
        wait_sc(p)


def kernel(x):
    xf = jnp.transpose(x, (1, 0, 2)).reshape(20 * B, D)
    mesh = plsc.VectorSubcoreMesh(core_axis_name="c", subcore_axis_name="s")
    run = functools.partial(
        pl.kernel,
        mesh=mesh,
        out_type=jax.ShapeDtypeStruct((B * R_OUT, D), jnp.float32),
        compiler_params=pltpu.CompilerParams(use_tc_tiling_on_sc=True),
        scratch_types=[
            [pltpu.VMEM((NROWS, D), jnp.float32) for _ in range(NSLOT)],
            [pltpu.VMEM((NROWS,), jnp.int32) for _ in range(NSLOT)],
            [pltpu.VMEM((NROWS,), jnp.int32) for _ in range(NSLOT)],
            pltpu.VMEM((NROWS,), jnp.int32),
            [pltpu.SemaphoreType.DMA for _ in range(NSLOT)],
            [pltpu.SemaphoreType.DMA for _ in range(NSLOT)],
        ],
    )(_body)
    out = run(xf)
    return out.reshape(B, 3, 2, D)
